# baseline (speedup 1.0000x reference)
import jax
import jax.numpy as jnp
from jax.experimental import pallas as pl
from jax.experimental.pallas import tpu as pltpu

_B, _H, _W, _CIN = 8, 224, 224, 96
_PH, _PW = 16, 16
_NPH, _NPW = 14, 14
_KOUT = 256
_M = _B * _PH * _PW
_MB = _PH * _PW

_CR = 32
_NCK = _H // _CR
_NX = _B * _NCK
_NBUF = 8

_CCK = 18816 // 14
_NC = 14
_CBUF = 2


def _reduce_kernel(x_hbm, c_hbm, o_ref, xbuf, cbuf, s_ref, w_ref, xsem, csem):
    s_ref[...] = jnp.zeros_like(s_ref)
    w_ref[...] = jnp.zeros_like(w_ref)

    def x_copy(k, slot):
        b = k // _NCK
        r = (k % _NCK) * _CR
        return pltpu.make_async_copy(
            x_hbm.at[b, pl.ds(r, _CR)], xbuf.at[slot], xsem.at[slot])

    def c_copy(k, slot):
        return pltpu.make_async_copy(
            c_hbm.at[pl.ds(k * _CCK, _CCK)], cbuf.at[slot], csem.at[slot])

    for k in range(_NBUF):
        x_copy(k, k).start()
    for k in range(_CBUF):
        c_copy(k, k).start()

    def c_body(k, carry):
        slot = k % _CBUF
        c_copy(k, slot).wait()
        cr = cbuf.at[slot]
        w = jnp.zeros((1, _KOUT), jnp.float32)
        for t in range(6):
            w = w + cr[pl.ds(224 * t, 224), :].sum(axis=0, keepdims=True)

        @pl.when(k + _CBUF < _NC)
        def _next():
            c_copy(k + _CBUF, slot).start()

        w_ref[...] += w
        return carry

    def x_body(k, carry):
        slot = k % _NBUF
        x_copy(k, slot).wait()
        xr = xbuf.at[slot]
        acc = xr[:, 0:_PW, :]
        for j in range(1, _NPW):
            acc = acc + xr[:, _PW * j:_PW * (j + 1), :]
        acc = acc[0:_PH] + acc[_PH:_CR]
        part = acc.reshape(_MB, _CIN).sum(axis=1, keepdims=True)

        @pl.when(k + _NBUF < _NX)
        def _next():
            x_copy(k + _NBUF, slot).start()

        b = k // _NCK
        s_ref[pl.ds(_MB * b, _MB), :] += part
        return carry

    jax.lax.fori_loop(0, _NC, c_body, 0, unroll=False)
    jax.lax.fori_loop(0, _NX, x_body, 0, unroll=False)
    for t in range(_B):
        o_ref[pl.ds(_MB * t, _MB), :] = (
            s_ref[pl.ds(_MB * t, _MB), :] * w_ref[...])


def kernel(x, C):
    c2 = C.reshape(18816, _KOUT)
    out2 = pl.pallas_call(
        _reduce_kernel,
        in_specs=[
            pl.BlockSpec(memory_space=pltpu.HBM),
            pl.BlockSpec(memory_space=pltpu.HBM),
        ],
        out_specs=pl.BlockSpec(memory_space=pltpu.MemorySpace.VMEM),
        out_shape=jax.ShapeDtypeStruct((_M, _KOUT), jnp.float32),
        scratch_shapes=[
            pltpu.VMEM((_NBUF, _CR, _W, _CIN), jnp.float32),
            pltpu.VMEM((_CBUF, _CCK, _KOUT), jnp.float32),
            pltpu.VMEM((_M, 1), jnp.float32),
            pltpu.VMEM((1, _KOUT), jnp.float32),
            pltpu.SemaphoreType.DMA((_NBUF,)),
            pltpu.SemaphoreType.DMA((_CBUF,)),
        ],
    )(x, c2)
    out = out2.reshape(_B, _PH, _PW, _KOUT).transpose(0, 3, 1, 2)
    return out.reshape(_B, _KOUT, 1, _PH, _PW)
